# baseline (speedup 1.0000x reference)
import functools

import jax
import jax.numpy as jnp
from jax.experimental import pallas as pl
from jax.experimental.pallas import tpu as pltpu


def _pos_kernel(col_ref, row_ref, out_ref, *, h, w):
    _, d = col_ref.shape
    hw = h * w

    kc = jax.lax.broadcasted_iota(jnp.int32, (w, hw), 1)
    sc = jax.lax.broadcasted_iota(jnp.int32, (w, hw), 0)
    kr = jax.lax.broadcasted_iota(jnp.int32, (h, hw), 1)
    sr = jax.lax.broadcasted_iota(jnp.int32, (h, hw), 0)
    sel_col = (kc % w == sc).astype(jnp.float32)
    sel_row = (kr // w == sr).astype(jnp.float32)

    col = col_ref[0:w, :]
    row = row_ref[0:h, :]
    dn = (((0,), (0,)), ((), ()))
    out_ref[0:d, :] = jax.lax.dot_general(
        col, sel_col, dn, preferred_element_type=jnp.float32)
    out_ref[d : 2 * d, :] = jax.lax.dot_general(
        row, sel_row, dn, preferred_element_type=jnp.float32)


def _bcast_kernel(pos_hbm, out_hbm, sems, *, B):
    for b in range(B):
        pltpu.make_async_copy(pos_hbm, out_hbm.at[b], sems.at[b]).start()
    for b in range(B):
        pltpu.make_async_copy(pos_hbm, out_hbm.at[b], sems.at[b]).wait()


def kernel(x, mask, row_embed, col_embed):
    B = x.shape[0]
    h, w = x.shape[-2], x.shape[-1]
    n, d = col_embed.shape

    pos2d = pl.pallas_call(
        functools.partial(_pos_kernel, h=h, w=w),
        out_shape=jax.ShapeDtypeStruct((2 * d, h * w), jnp.float32),
    )(col_embed, row_embed)
    pos3d = pos2d.reshape(2 * d, h, w)

    return pl.pallas_call(
        functools.partial(_bcast_kernel, B=B),
        in_specs=[pl.BlockSpec(memory_space=pl.ANY)],
        out_specs=pl.BlockSpec(memory_space=pl.ANY),
        out_shape=jax.ShapeDtypeStruct((B, 2 * d, h, w), jnp.float32),
        scratch_shapes=[pltpu.SemaphoreType.DMA((B,))],
    )(pos3d)


# speedup vs baseline: 86.9761x; 86.9761x over previous
import functools

import jax
import jax.numpy as jnp
from jax.experimental import pallas as pl
from jax.experimental.pallas import tpu as pltpu


def _pos_kernel(col_ref, row_ref, out_hbm, scratch, sems, *, h, w, B):
    _, d = col_ref.shape
    hw = h * w

    kc = jax.lax.broadcasted_iota(jnp.int32, (w, hw), 1)
    sc = jax.lax.broadcasted_iota(jnp.int32, (w, hw), 0)
    kr = jax.lax.broadcasted_iota(jnp.int32, (h, hw), 1)
    sr = jax.lax.broadcasted_iota(jnp.int32, (h, hw), 0)
    sel_col = (kc % w == sc).astype(jnp.float32)
    sel_row = (kr // w == sr).astype(jnp.float32)

    col = col_ref[0:w, :]
    row = row_ref[0:h, :]
    dn = (((0,), (0,)), ((), ()))
    scratch[0:d, :] = jax.lax.dot_general(
        col, sel_col, dn, preferred_element_type=jnp.float32)
    scratch[d : 2 * d, :] = jax.lax.dot_general(
        row, sel_row, dn, preferred_element_type=jnp.float32)

    for b in range(B):
        pltpu.make_async_copy(scratch, out_hbm.at[b], sems.at[b]).start()
    for b in range(B):
        pltpu.make_async_copy(scratch, out_hbm.at[b], sems.at[b]).wait()


def kernel(x, mask, row_embed, col_embed):
    B = x.shape[0]
    h, w = x.shape[-2], x.shape[-1]
    n, d = col_embed.shape

    out = pl.pallas_call(
        functools.partial(_pos_kernel, h=h, w=w, B=B),
        in_specs=[
            pl.BlockSpec(memory_space=pltpu.MemorySpace.VMEM),
            pl.BlockSpec(memory_space=pltpu.MemorySpace.VMEM),
        ],
        out_specs=pl.BlockSpec(memory_space=pl.ANY),
        out_shape=jax.ShapeDtypeStruct((B, 2 * d, h * w), jnp.float32),
        scratch_shapes=[
            pltpu.VMEM((2 * d, h * w), jnp.float32),
            pltpu.SemaphoreType.DMA((B,)),
        ],
    )(col_embed, row_embed)
    return out.reshape(B, 2 * d, h, w)
